# Initial kernel scaffold; baseline (speedup 1.0000x reference)
import jax
import jax.numpy as jnp
from jax.experimental import pallas as pl


def kernel(x, Wr, br, W1, b1, W2, b2):
    raise NotImplementedError("write your pallas kernel here")


# speedup vs baseline: 3.0373x; 3.0373x over previous
import functools

import jax
import jax.numpy as jnp
from jax.experimental import pallas as pl
from jax.experimental.pallas import tpu as pltpu

DIM = 1024
N_EXPERTS = 8
TOPK = 2
N_TOKENS = 2048
HID = 4 * DIM
HBLK = 512
N_HBLK = HID // HBLK


def _moe_dense_body(x_ref, wr_ref, br_ref, w1_ref, b1_ref, w2_ref, b2_ref,
                    out_ref, coef_ref):
    e = pl.program_id(0)
    h = pl.program_id(1)

    @pl.when(jnp.logical_and(e == 0, h == 0))
    def _router():
        x = x_ref[...]
        logits = jnp.dot(x, wr_ref[...], preferred_element_type=jnp.float32)
        logits = logits + br_ref[...]
        e_ids = jax.lax.broadcasted_iota(jnp.int32, logits.shape, 1)
        m1 = jnp.max(logits, axis=1, keepdims=True)
        idx1 = jnp.min(jnp.where(logits == m1, e_ids, N_EXPERTS), axis=1,
                       keepdims=True)
        masked = jnp.where(e_ids == idx1, -jnp.inf, logits)
        m2 = jnp.max(masked, axis=1, keepdims=True)
        idx2 = jnp.min(jnp.where(masked == m2, e_ids, N_EXPERTS), axis=1,
                       keepdims=True)
        t = jnp.exp(m2 - m1)
        s = 1.0 + t
        w1v = 1.0 / s
        w2v = t / s
        coef = (jnp.where(e_ids == idx1, w1v, 0.0)
                + jnp.where(e_ids == idx2, w2v, 0.0))
        coef_ref[...] = coef
        out_ref[...] = jnp.dot(coef, b2_ref[...],
                               preferred_element_type=jnp.float32)

    x = x_ref[...]
    hpre = jnp.dot(x, w1_ref[0], preferred_element_type=jnp.float32)
    hpre = hpre + b1_ref[0]
    hact = hpre * 0.5 * (1.0 + jax.lax.erf(hpre * 0.7071067811865476))
    part = jnp.dot(hact, w2_ref[0], preferred_element_type=jnp.float32)
    c = coef_ref[...]
    e_ids = jax.lax.broadcasted_iota(jnp.int32, c.shape, 1)
    c_e = jnp.sum(jnp.where(e_ids == e, c, 0.0), axis=1, keepdims=True)
    out_ref[...] += part * c_e


def kernel(x, Wr, br, W1, b1, W2, b2):
    br2 = br.reshape(1, N_EXPERTS)
    grid = (N_EXPERTS, N_HBLK)
    out = pl.pallas_call(
        _moe_dense_body,
        grid=grid,
        in_specs=[
            pl.BlockSpec((N_TOKENS, DIM), lambda e, h: (0, 0)),
            pl.BlockSpec((DIM, N_EXPERTS), lambda e, h: (0, 0)),
            pl.BlockSpec((1, N_EXPERTS), lambda e, h: (0, 0)),
            pl.BlockSpec((1, DIM, HBLK), lambda e, h: (e, 0, h)),
            pl.BlockSpec((1, 1, HBLK), lambda e, h: (e, 0, h)),
            pl.BlockSpec((1, HBLK, DIM), lambda e, h: (e, h, 0)),
            pl.BlockSpec((N_EXPERTS, DIM), lambda e, h: (0, 0)),
        ],
        out_specs=pl.BlockSpec((N_TOKENS, DIM), lambda e, h: (0, 0)),
        out_shape=jax.ShapeDtypeStruct((N_TOKENS, DIM), jnp.float32),
        scratch_shapes=[pltpu.VMEM((N_TOKENS, N_EXPERTS), jnp.float32)],
    )(x, Wr, br2, W1, b1.reshape(N_EXPERTS, 1, HID), W2, b2)
    return out
